# baseline (speedup 1.0000x reference)
import functools

import jax
import jax.numpy as jnp
from jax import lax
from jax.experimental import pallas as pl
from jax.experimental.pallas import tpu as pltpu
from jax.experimental.pallas import tpu_sc as plsc

B, P, R, D, T = 1024, 20, 8, 128, 10000
N = B * P
NC, NS, L = 2, 16, 16
NW = NC * NS
ROWS_W = N // NW
CH = 16
NCHUNK = ROWS_W // CH
NPAIR = NCHUNK // 2
GATH = CH * R
TW = D // 2
KB = D // (2 * L)
NEG_BIG = -1e30


def _han_body(x_hbm, t32_hbm, refidx_hbm, titleidx_hbm, endyr_hbm,
              out_hbm,
              refidx_v, titleidx_v, endyr_v,
              focal_v, ref_v, title_v, outb_v,
              sem_r0, sem_r1, sem_t0, sem_t1, sem_f0, sem_f1,
              sem_o0, sem_o1):
  cid = lax.axis_index("c")
  sid = lax.axis_index("s")
  wid = sid * NC + cid
  row0 = wid * ROWS_W

  pltpu.sync_copy(refidx_hbm.at[pl.ds(row0 * R, ROWS_W * R)], refidx_v)
  pltpu.sync_copy(titleidx_hbm.at[pl.ds(row0 * R, ROWS_W * R)], titleidx_v)
  pltpu.sync_copy(endyr_hbm.at[pl.ds(row0, ROWS_W)], endyr_v)

  lane = lax.iota(jnp.int32, L)
  lane2 = lane * 2

  sem_r = (sem_r0, sem_r1)
  sem_t = (sem_t0, sem_t1)
  sem_f = (sem_f0, sem_f1)
  sem_o = (sem_o0, sem_o1)

  def gather_ops(ci, p):
    base = row0 + ci * CH
    return (
        (x_hbm.at[refidx_v.at[pl.ds(ci * GATH, GATH)]],
         ref_v.at[pl.ds(p * GATH, GATH)], sem_r[p]),
        (t32_hbm.at[titleidx_v.at[pl.ds(ci * GATH, GATH)]],
         title_v.at[pl.ds(p * GATH, GATH)], sem_t[p]),
        (x_hbm.at[pl.ds(base, CH)],
         focal_v.at[pl.ds(p * CH, CH)], sem_f[p]),
    )

  def issue(ci, p):
    for a in gather_ops(ci, p):
      pltpu.async_copy(*a)

  def wait_gathers(ci, p):
    for a in gather_ops(ci, p):
      pltpu.make_async_copy(*a).wait()

  def out_op(ci, p):
    base = row0 + ci * CH
    return (outb_v.at[pl.ds(p * CH, CH)], out_hbm.at[pl.ds(base, CH)],
            sem_o[p])

  def compute(ci, p):
    def row_body(i):
      ib = p * CH + i
      rbase = p * GATH + i * R
      f = [focal_v[ib, pl.ds(k * L, L)] for k in range(D // L)]
      sv = jnp.full((L,), NEG_BIG, jnp.float32)
      for r in range(R):
        acc_a = f[0] * ref_v[rbase + r, pl.ds(0, L)]
        acc_b = f[1] * ref_v[rbase + r, pl.ds(L, L)]
        for k in range(2, D // L, 2):
          acc_a = acc_a + f[k] * ref_v[rbase + r, pl.ds(k * L, L)]
          acc_b = acc_b + f[k + 1] * ref_v[rbase + r, pl.ds((k + 1) * L, L)]
        sv = jnp.where(lane == r, jnp.sum(acc_a + acc_b), sv)
      e = jnp.exp(sv)
      denom = jnp.sum(e)
      eyv = plsc.load_gather(
          endyr_v, [jnp.broadcast_to(ci * CH + i, (L,)).astype(jnp.int32)])
      mval = jnp.where(eyv != 0, 1.0, 0.0)
      sim = (e / denom) * mval
      w = [sim[r] for r in range(R)]
      for k in range(D // L):
        outb_v[ib, pl.ds(k * L, L)] = f[k]
      for k in range(D // L):
        ga = w[0] * title_v[rbase, pl.ds(k * L, L)]
        gb = w[1] * title_v[rbase + 1, pl.ds(k * L, L)]
        for r in range(2, R, 2):
          ga = ga + w[r] * title_v[rbase + r, pl.ds(k * L, L)]
          gb = gb + w[r + 1] * title_v[rbase + r + 1, pl.ds(k * L, L)]
        outb_v[ib, pl.ds(D + k * L, L)] = ga + gb

    plsc.parallel_loop(0, CH, unroll=4)(row_body)

  issue(0, 0)

  def pair_body(cj, carry):
    ci0 = 2 * cj
    issue(ci0 + 1, 1)
    wait_gathers(ci0, 0)

    @pl.when(cj > 0)
    def _():
      pltpu.make_async_copy(*out_op(ci0 - 2, 0)).wait()

    compute(ci0, 0)
    pltpu.async_copy(*out_op(ci0, 0))

    @pl.when(cj < NPAIR - 1)
    def _():
      issue(ci0 + 2, 0)

    wait_gathers(ci0 + 1, 1)

    @pl.when(cj > 0)
    def _():
      pltpu.make_async_copy(*out_op(ci0 - 1, 1)).wait()

    compute(ci0 + 1, 1)
    pltpu.async_copy(*out_op(ci0 + 1, 1))
    return carry

  lax.fori_loop(0, NPAIR, pair_body, 0)
  pltpu.make_async_copy(*out_op(NCHUNK - 2, 0)).wait()
  pltpu.make_async_copy(*out_op(NCHUNK - 1, 1)).wait()


_han_sc = functools.partial(
    pl.kernel,
    mesh=plsc.VectorSubcoreMesh(core_axis_name="c", subcore_axis_name="s"),
    out_type=jax.ShapeDtypeStruct((N, 2 * D), jnp.float32),
    compiler_params=pltpu.CompilerParams(
        needs_layout_passes=False, use_tc_tiling_on_sc=False),
    scratch_types=[
        pltpu.VMEM((ROWS_W * R,), jnp.int32),
        pltpu.VMEM((ROWS_W * R,), jnp.int32),
        pltpu.VMEM((ROWS_W,), jnp.int32),
        pltpu.VMEM((2 * CH, D), jnp.float32),
        pltpu.VMEM((2 * GATH, D), jnp.float32),
        pltpu.VMEM((2 * GATH, D), jnp.float32),
        pltpu.VMEM((2 * CH, 2 * D), jnp.float32),
        pltpu.SemaphoreType.DMA,
        pltpu.SemaphoreType.DMA,
        pltpu.SemaphoreType.DMA,
        pltpu.SemaphoreType.DMA,
        pltpu.SemaphoreType.DMA,
        pltpu.SemaphoreType.DMA,
        pltpu.SemaphoreType.DMA,
        pltpu.SemaphoreType.DMA,
    ],
)(_han_body)


def kernel(title_emb_mat, emp_ids, end_yrs, batch_label, inputs,
           ref_batch_pos, ref_job_idx, ref_title_idx):
  del emp_ids, batch_label
  x = inputs.reshape(N, D)
  refidx = (ref_batch_pos.astype(jnp.int32) * P
            + ref_job_idx.astype(jnp.int32)).reshape(N * R)
  titleidx = ref_title_idx.astype(jnp.int32).reshape(N * R)
  endyr = end_yrs.astype(jnp.int32).reshape(N)
  return _han_sc(x, title_emb_mat, refidx, titleidx, endyr)


# speedup vs baseline: 1.1704x; 1.1704x over previous
import functools

import jax
import jax.numpy as jnp
from jax import lax
from jax.experimental import pallas as pl
from jax.experimental.pallas import tpu as pltpu
from jax.experimental.pallas import tpu_sc as plsc

B, P, R, D, T = 1024, 20, 8, 128, 10000
N = B * P
NC, NS, L = 2, 16, 16
NW = NC * NS
ROWS_W = N // NW
CH = 16
NCHUNK = ROWS_W // CH
NPAIR = NCHUNK // 2
GATH = CH * R
TW = D // 2
KB = D // (2 * L)
NEG_BIG = -1e30


def _han_body(x_hbm, t32_hbm, refidx_hbm, titleidx_hbm, endyr_hbm,
              out_hbm,
              refidx_v, titleidx_v, endyr_v,
              focal_v, ref_v, title_v, outb_v,
              sem_r0, sem_r1, sem_t0, sem_t1, sem_f0, sem_f1,
              sem_o0, sem_o1):
  cid = lax.axis_index("c")
  sid = lax.axis_index("s")
  wid = sid * NC + cid
  row0 = wid * ROWS_W

  pltpu.sync_copy(refidx_hbm.at[pl.ds(row0 * R, ROWS_W * R)], refidx_v)
  pltpu.sync_copy(titleidx_hbm.at[pl.ds(row0 * R, ROWS_W * R)], titleidx_v)
  pltpu.sync_copy(endyr_hbm.at[pl.ds(row0, ROWS_W)], endyr_v)

  lane = lax.iota(jnp.int32, L)
  lane2 = lane * 2

  sem_r = (sem_r0, sem_r1)
  sem_t = (sem_t0, sem_t1)
  sem_f = (sem_f0, sem_f1)
  sem_o = (sem_o0, sem_o1)

  def gather_ops(ci, p):
    base = row0 + ci * CH
    return (
        (x_hbm.at[refidx_v.at[pl.ds(ci * GATH, GATH)]],
         ref_v.at[pl.ds(p * GATH, GATH)], sem_r[p]),
        (t32_hbm.at[titleidx_v.at[pl.ds(ci * GATH, GATH)]],
         title_v.at[pl.ds(p * GATH, GATH)], sem_t[p]),
        (x_hbm.at[pl.ds(base, CH)],
         focal_v.at[pl.ds(p * CH, CH)], sem_f[p]),
    )

  def issue(ci, p):
    for a in gather_ops(ci, p):
      pltpu.async_copy(*a)

  def wait_gathers(ci, p):
    for a in gather_ops(ci, p):
      pltpu.make_async_copy(*a).wait()

  def out_op(ci, p):
    base = row0 + ci * CH
    return (outb_v.at[pl.ds(p * CH, CH)], out_hbm.at[pl.ds(base, CH)],
            sem_o[p])

  def compute(ci, p):
    def row_body(i):
      ib = p * CH + i
      rbase = p * GATH + i * R
      f = [focal_v[ib, pl.ds(k * L, L)] for k in range(D // L)]
      sv = jnp.full((L,), NEG_BIG, jnp.float32)
      for r in range(R):
        acc_a = f[0] * ref_v[rbase + r, pl.ds(0, L)]
        acc_b = f[1] * ref_v[rbase + r, pl.ds(L, L)]
        for k in range(2, D // L, 2):
          acc_a = acc_a + f[k] * ref_v[rbase + r, pl.ds(k * L, L)]
          acc_b = acc_b + f[k + 1] * ref_v[rbase + r, pl.ds((k + 1) * L, L)]
        sv = jnp.where(lane == r, jnp.sum(acc_a + acc_b), sv)
      e = jnp.exp(sv)
      denom = jnp.sum(e)
      eyv = plsc.load_gather(
          endyr_v, [jnp.broadcast_to(ci * CH + i, (L,)).astype(jnp.int32)])
      mval = jnp.where(eyv != 0, 1.0, 0.0)
      sim = (e / denom) * mval
      w = [sim[r] for r in range(R)]
      for k in range(D // L):
        outb_v[ib, pl.ds(k * L, L)] = f[k]
      for k in range(D // L):
        ga = w[0] * title_v[rbase, pl.ds(k * L, L)]
        gb = w[1] * title_v[rbase + 1, pl.ds(k * L, L)]
        for r in range(2, R, 2):
          ga = ga + w[r] * title_v[rbase + r, pl.ds(k * L, L)]
          gb = gb + w[r + 1] * title_v[rbase + r + 1, pl.ds(k * L, L)]
        outb_v[ib, pl.ds(D + k * L, L)] = ga + gb

    plsc.parallel_loop(0, CH, unroll=2)(row_body)

  issue(0, 0)

  def pair_body(cj, carry):
    ci0 = 2 * cj
    issue(ci0 + 1, 1)
    wait_gathers(ci0, 0)

    @pl.when(cj > 0)
    def _():
      pltpu.make_async_copy(*out_op(ci0 - 2, 0)).wait()

    compute(ci0, 0)
    pltpu.async_copy(*out_op(ci0, 0))

    @pl.when(cj < NPAIR - 1)
    def _():
      issue(ci0 + 2, 0)

    wait_gathers(ci0 + 1, 1)

    @pl.when(cj > 0)
    def _():
      pltpu.make_async_copy(*out_op(ci0 - 1, 1)).wait()

    compute(ci0 + 1, 1)
    pltpu.async_copy(*out_op(ci0 + 1, 1))
    return carry

  lax.fori_loop(0, NPAIR, pair_body, 0)
  pltpu.make_async_copy(*out_op(NCHUNK - 2, 0)).wait()
  pltpu.make_async_copy(*out_op(NCHUNK - 1, 1)).wait()


_han_sc = functools.partial(
    pl.kernel,
    mesh=plsc.VectorSubcoreMesh(core_axis_name="c", subcore_axis_name="s"),
    out_type=jax.ShapeDtypeStruct((N, 2 * D), jnp.float32),
    compiler_params=pltpu.CompilerParams(needs_layout_passes=False),
    scratch_types=[
        pltpu.VMEM((ROWS_W * R,), jnp.int32),
        pltpu.VMEM((ROWS_W * R,), jnp.int32),
        pltpu.VMEM((ROWS_W,), jnp.int32),
        pltpu.VMEM((2 * CH, D), jnp.float32),
        pltpu.VMEM((2 * GATH, D), jnp.float32),
        pltpu.VMEM((2 * GATH, D), jnp.float32),
        pltpu.VMEM((2 * CH, 2 * D), jnp.float32),
        pltpu.SemaphoreType.DMA,
        pltpu.SemaphoreType.DMA,
        pltpu.SemaphoreType.DMA,
        pltpu.SemaphoreType.DMA,
        pltpu.SemaphoreType.DMA,
        pltpu.SemaphoreType.DMA,
        pltpu.SemaphoreType.DMA,
        pltpu.SemaphoreType.DMA,
    ],
)(_han_body)


def kernel(title_emb_mat, emp_ids, end_yrs, batch_label, inputs,
           ref_batch_pos, ref_job_idx, ref_title_idx):
  del emp_ids, batch_label
  x = inputs.reshape(N, D)
  refidx = (ref_batch_pos.astype(jnp.int32) * P
            + ref_job_idx.astype(jnp.int32)).reshape(N * R)
  titleidx = ref_title_idx.astype(jnp.int32).reshape(N * R)
  endyr = end_yrs.astype(jnp.int32).reshape(N)
  return _han_sc(x, title_emb_mat, refidx, titleidx, endyr)


# speedup vs baseline: 1.1757x; 1.0045x over previous
import functools

import jax
import jax.numpy as jnp
from jax import lax
from jax.experimental import pallas as pl
from jax.experimental.pallas import tpu as pltpu
from jax.experimental.pallas import tpu_sc as plsc

B, P, R, D, T = 1024, 20, 8, 128, 10000
N = B * P
NC, NS, L = 2, 16, 16
NW = NC * NS
ROWS_W = N // NW
CH = 16
NCHUNK = ROWS_W // CH
NPAIR = NCHUNK // 2
GATH = CH * R
TW = D // 2
KB = D // (2 * L)
NEG_BIG = -1e30


def _han_body(x_hbm, t32_hbm, refidx_hbm, titleidx_hbm, endyr_hbm,
              out_hbm,
              refidx_v, titleidx_v, endyr_v,
              focal_v, ref_v, title_v, outb_v,
              sem_r0, sem_r1, sem_t0, sem_t1, sem_f0, sem_f1,
              sem_o0, sem_o1):
  cid = lax.axis_index("c")
  sid = lax.axis_index("s")
  wid = sid * NC + cid
  row0 = wid * ROWS_W

  pltpu.sync_copy(refidx_hbm.at[pl.ds(row0 * R, ROWS_W * R)], refidx_v)
  pltpu.sync_copy(titleidx_hbm.at[pl.ds(row0 * R, ROWS_W * R)], titleidx_v)
  pltpu.sync_copy(endyr_hbm.at[pl.ds(row0, ROWS_W)], endyr_v)

  lane = lax.iota(jnp.int32, L)
  lane2 = lane * 2

  sem_r = (sem_r0, sem_r1)
  sem_t = (sem_t0, sem_t1)
  sem_f = (sem_f0, sem_f1)
  sem_o = (sem_o0, sem_o1)

  def gather_ops(ci, p):
    base = row0 + ci * CH
    return (
        (x_hbm.at[refidx_v.at[pl.ds(ci * GATH, GATH)]],
         ref_v.at[pl.ds(p * GATH, GATH)], sem_r[p]),
        (t32_hbm.at[titleidx_v.at[pl.ds(ci * GATH, GATH)]],
         title_v.at[pl.ds(p * GATH, GATH)], sem_t[p]),
        (x_hbm.at[pl.ds(base, CH)],
         focal_v.at[pl.ds(p * CH, CH)], sem_f[p]),
    )

  def issue(ci, p):
    for a in gather_ops(ci, p):
      pltpu.async_copy(*a)

  def wait_gathers(ci, p):
    for a in gather_ops(ci, p):
      pltpu.make_async_copy(*a).wait()

  def out_op(ci, p):
    base = row0 + ci * CH
    return (outb_v.at[pl.ds(p * CH, CH)], out_hbm.at[pl.ds(base, CH)],
            sem_o[p])

  def compute(ci, p):
    def row_body(i):
      ib = p * CH + i
      rbase = p * GATH + i * R
      f = [focal_v[ib, pl.ds(k * L, L)] for k in range(D // L)]
      sv = jnp.full((L,), NEG_BIG, jnp.float32)
      for r in range(R):
        acc_a = f[0] * ref_v[rbase + r, pl.ds(0, L)]
        acc_b = f[1] * ref_v[rbase + r, pl.ds(L, L)]
        for k in range(2, D // L, 2):
          acc_a = acc_a + f[k] * ref_v[rbase + r, pl.ds(k * L, L)]
          acc_b = acc_b + f[k + 1] * ref_v[rbase + r, pl.ds((k + 1) * L, L)]
        sv = jnp.where(lane == r, jnp.sum(acc_a + acc_b), sv)
      e = jnp.exp(sv)
      denom = jnp.sum(e)
      eyv = plsc.load_gather(
          endyr_v, [jnp.broadcast_to(ci * CH + i, (L,)).astype(jnp.int32)])
      mval = jnp.where(eyv != 0, 1.0, 0.0)
      sim = (e / denom) * mval
      w = [sim[r] for r in range(R)]
      for k in range(D // L):
        outb_v[ib, pl.ds(k * L, L)] = f[k]
      for k in range(D // L):
        ga = w[0] * title_v[rbase, pl.ds(k * L, L)]
        gb = w[1] * title_v[rbase + 1, pl.ds(k * L, L)]
        for r in range(2, R, 2):
          ga = ga + w[r] * title_v[rbase + r, pl.ds(k * L, L)]
          gb = gb + w[r + 1] * title_v[rbase + r + 1, pl.ds(k * L, L)]
        outb_v[ib, pl.ds(D + k * L, L)] = ga + gb

    plsc.parallel_loop(0, CH, unroll=3)(row_body)

  issue(0, 0)

  def pair_body(cj, carry):
    ci0 = 2 * cj
    issue(ci0 + 1, 1)
    wait_gathers(ci0, 0)

    @pl.when(cj > 0)
    def _():
      pltpu.make_async_copy(*out_op(ci0 - 2, 0)).wait()

    compute(ci0, 0)
    pltpu.async_copy(*out_op(ci0, 0))

    @pl.when(cj < NPAIR - 1)
    def _():
      issue(ci0 + 2, 0)

    wait_gathers(ci0 + 1, 1)

    @pl.when(cj > 0)
    def _():
      pltpu.make_async_copy(*out_op(ci0 - 1, 1)).wait()

    compute(ci0 + 1, 1)
    pltpu.async_copy(*out_op(ci0 + 1, 1))
    return carry

  lax.fori_loop(0, NPAIR, pair_body, 0)
  pltpu.make_async_copy(*out_op(NCHUNK - 2, 0)).wait()
  pltpu.make_async_copy(*out_op(NCHUNK - 1, 1)).wait()


_han_sc = functools.partial(
    pl.kernel,
    mesh=plsc.VectorSubcoreMesh(core_axis_name="c", subcore_axis_name="s"),
    out_type=jax.ShapeDtypeStruct((N, 2 * D), jnp.float32),
    compiler_params=pltpu.CompilerParams(needs_layout_passes=False),
    scratch_types=[
        pltpu.VMEM((ROWS_W * R,), jnp.int32),
        pltpu.VMEM((ROWS_W * R,), jnp.int32),
        pltpu.VMEM((ROWS_W,), jnp.int32),
        pltpu.VMEM((2 * CH, D), jnp.float32),
        pltpu.VMEM((2 * GATH, D), jnp.float32),
        pltpu.VMEM((2 * GATH, D), jnp.float32),
        pltpu.VMEM((2 * CH, 2 * D), jnp.float32),
        pltpu.SemaphoreType.DMA,
        pltpu.SemaphoreType.DMA,
        pltpu.SemaphoreType.DMA,
        pltpu.SemaphoreType.DMA,
        pltpu.SemaphoreType.DMA,
        pltpu.SemaphoreType.DMA,
        pltpu.SemaphoreType.DMA,
        pltpu.SemaphoreType.DMA,
    ],
)(_han_body)


def kernel(title_emb_mat, emp_ids, end_yrs, batch_label, inputs,
           ref_batch_pos, ref_job_idx, ref_title_idx):
  del emp_ids, batch_label
  x = inputs.reshape(N, D)
  refidx = (ref_batch_pos.astype(jnp.int32) * P
            + ref_job_idx.astype(jnp.int32)).reshape(N * R)
  titleidx = ref_title_idx.astype(jnp.int32).reshape(N * R)
  endyr = end_yrs.astype(jnp.int32).reshape(N)
  return _han_sc(x, title_emb_mat, refidx, titleidx, endyr)


# speedup vs baseline: 1.3164x; 1.1197x over previous
import functools

import jax
import jax.numpy as jnp
from jax import lax
from jax.experimental import pallas as pl
from jax.experimental.pallas import tpu as pltpu
from jax.experimental.pallas import tpu_sc as plsc

B, P, R, D, T = 1024, 20, 8, 128, 10000
N = B * P
NC, NS, L = 2, 16, 16
NW = NC * NS
ROWS_W = N // NW
CH = 16
NCHUNK = ROWS_W // CH
NPAIR = NCHUNK // 2
GATH = CH * R
TW = D // 2
KB = D // (2 * L)
NEG_BIG = -1e30


def _han_body(x_hbm, t_hbm, packed_hbm,
              out_hbm,
              packed_v, refidx_v, titleidx_v,
              focal_v, ref_v, title_v, outb_v,
              sem_r0, sem_r1, sem_t0, sem_t1, sem_f0, sem_f1,
              sem_o0, sem_o1):
  cid = lax.axis_index("c")
  sid = lax.axis_index("s")
  wid = sid * NC + cid
  row0 = wid * ROWS_W

  pltpu.sync_copy(packed_hbm.at[pl.ds(row0 * R, ROWS_W * R)], packed_v)

  @plsc.parallel_loop(0, ROWS_W * R // L, unroll=4)
  def _unpack_idx(j):
    v = packed_v[pl.ds(j * L, L)]
    refidx_v[pl.ds(j * L, L)] = v & 0x7FFF
    titleidx_v[pl.ds(j * L, L)] = (v >> 15) & 0x3FFF

  lane = lax.iota(jnp.int32, L)

  sem_r = (sem_r0, sem_r1)
  sem_t = (sem_t0, sem_t1)
  sem_f = (sem_f0, sem_f1)
  sem_o = (sem_o0, sem_o1)

  def gather_ops(ci, p):
    base = row0 + ci * CH
    return (
        (x_hbm.at[refidx_v.at[pl.ds(ci * GATH, GATH)]],
         ref_v.at[pl.ds(p * GATH, GATH)], sem_r[p]),
        (t_hbm.at[titleidx_v.at[pl.ds(ci * GATH, GATH)]],
         title_v.at[pl.ds(p * GATH, GATH)], sem_t[p]),
        (x_hbm.at[pl.ds(base, CH)],
         focal_v.at[pl.ds(p * CH, CH)], sem_f[p]),
    )

  def issue(ci, p):
    for a in gather_ops(ci, p):
      pltpu.async_copy(*a)

  def wait_gathers(ci, p):
    for a in gather_ops(ci, p):
      pltpu.make_async_copy(*a).wait()

  def out_op(ci, p):
    base = row0 + ci * CH
    return (outb_v.at[pl.ds(p * CH, CH)], out_hbm.at[pl.ds(base, CH)],
            sem_o[p])

  def compute(ci, p):
    def row_body(i):
      ib = p * CH + i
      rbase = p * GATH + i * R
      f = [focal_v[ib, pl.ds(k * L, L)] for k in range(D // L)]
      sv = jnp.full((L,), NEG_BIG, jnp.float32)
      for r in range(R):
        acc_a = f[0] * ref_v[rbase + r, pl.ds(0, L)]
        acc_b = f[1] * ref_v[rbase + r, pl.ds(L, L)]
        for k in range(2, D // L, 2):
          acc_a = acc_a + f[k] * ref_v[rbase + r, pl.ds(k * L, L)]
          acc_b = acc_b + f[k + 1] * ref_v[rbase + r, pl.ds((k + 1) * L, L)]
        sv = jnp.where(lane == r, jnp.sum(acc_a + acc_b), sv)
      e = jnp.exp(sv)
      denom = jnp.sum(e)
      eyv = plsc.load_gather(
          packed_v,
          [jnp.broadcast_to((ci * CH + i) * R, (L,)).astype(jnp.int32)])
      mval = jnp.where((eyv >> 29) != 0, 1.0, 0.0)
      sim = (e / denom) * mval
      w = [sim[r] for r in range(R)]
      for k in range(D // L):
        outb_v[ib, pl.ds(k * L, L)] = f[k]
      for k in range(D // L):
        ga = w[0] * title_v[rbase, pl.ds(k * L, L)]
        gb = w[1] * title_v[rbase + 1, pl.ds(k * L, L)]
        for r in range(2, R, 2):
          ga = ga + w[r] * title_v[rbase + r, pl.ds(k * L, L)]
          gb = gb + w[r + 1] * title_v[rbase + r + 1, pl.ds(k * L, L)]
        outb_v[ib, pl.ds(D + k * L, L)] = ga + gb

    plsc.parallel_loop(0, CH, unroll=3)(row_body)

  issue(0, 0)

  def pair_body(cj, carry):
    ci0 = 2 * cj
    issue(ci0 + 1, 1)
    wait_gathers(ci0, 0)

    @pl.when(cj > 0)
    def _():
      pltpu.make_async_copy(*out_op(ci0 - 2, 0)).wait()

    compute(ci0, 0)
    pltpu.async_copy(*out_op(ci0, 0))

    @pl.when(cj < NPAIR - 1)
    def _():
      issue(ci0 + 2, 0)

    wait_gathers(ci0 + 1, 1)

    @pl.when(cj > 0)
    def _():
      pltpu.make_async_copy(*out_op(ci0 - 1, 1)).wait()

    compute(ci0 + 1, 1)
    pltpu.async_copy(*out_op(ci0 + 1, 1))
    return carry

  lax.fori_loop(0, NPAIR, pair_body, 0)
  pltpu.make_async_copy(*out_op(NCHUNK - 2, 0)).wait()
  pltpu.make_async_copy(*out_op(NCHUNK - 1, 1)).wait()


_han_sc = functools.partial(
    pl.kernel,
    mesh=plsc.VectorSubcoreMesh(core_axis_name="c", subcore_axis_name="s"),
    out_type=jax.ShapeDtypeStruct((N, 2 * D), jnp.float32),
    compiler_params=pltpu.CompilerParams(needs_layout_passes=False),
    scratch_types=[
        pltpu.VMEM((ROWS_W * R,), jnp.int32),
        pltpu.VMEM((ROWS_W * R,), jnp.int32),
        pltpu.VMEM((ROWS_W * R,), jnp.int32),
        pltpu.VMEM((2 * CH, D), jnp.float32),
        pltpu.VMEM((2 * GATH, D), jnp.float32),
        pltpu.VMEM((2 * GATH, D), jnp.float32),
        pltpu.VMEM((2 * CH, 2 * D), jnp.float32),
        pltpu.SemaphoreType.DMA,
        pltpu.SemaphoreType.DMA,
        pltpu.SemaphoreType.DMA,
        pltpu.SemaphoreType.DMA,
        pltpu.SemaphoreType.DMA,
        pltpu.SemaphoreType.DMA,
        pltpu.SemaphoreType.DMA,
        pltpu.SemaphoreType.DMA,
    ],
)(_han_body)


def kernel(title_emb_mat, emp_ids, end_yrs, batch_label, inputs,
           ref_batch_pos, ref_job_idx, ref_title_idx):
  del emp_ids, batch_label
  x = inputs.reshape(N, D)
  refidx = (ref_batch_pos.astype(jnp.int32) * P
            + ref_job_idx.astype(jnp.int32))
  packed = (refidx
            | (ref_title_idx.astype(jnp.int32) << 15)
            | ((end_yrs != 0).astype(jnp.int32)[:, :, None] << 29))
  return _han_sc(x, title_emb_mat, packed.reshape(N * R))
